# baseline (device time: 26248 ns/iter reference)
import jax
import jax.numpy as jnp
from jax import lax
from jax.experimental import pallas as pl
from jax.experimental.pallas import tpu as pltpu

N_DEV = 4


def kernel(x, Wq, K_ext, V_ext, Wo):
    B, Sq, E = x.shape
    _, Skv_sh, Hq, Dh = K_ext.shape
    Skv = Skv_sh * N_DEV
    Do = Hq * Dh

    x_b = x.astype(jnp.bfloat16)
    wq_r = Wq.reshape(E, Hq, Dh).transpose(1, 0, 2).astype(jnp.bfloat16)
    wo_b = Wo.astype(jnp.bfloat16)
    kt = K_ext.transpose(0, 2, 3, 1).astype(jnp.bfloat16)
    vt = V_ext.transpose(0, 2, 3, 1).astype(jnp.bfloat16)
    kv = jnp.stack([kt, vt], axis=0)

    def body(x_ref, wq_ref, kv_ref, wo_ref, out_ref, kv_full, send_sems, recv_sems):
        my_pos = lax.axis_index("i")
        right = lax.rem(my_pos + 1, N_DEV)
        left = lax.rem(my_pos + N_DEV - 1, N_DEV)

        barrier_sem = pltpu.get_barrier_semaphore()
        for nbr in (left, right):
            pl.semaphore_signal(
                barrier_sem, inc=1,
                device_id=(nbr,), device_id_type=pl.DeviceIdType.MESH,
            )
        pl.semaphore_wait(barrier_sem, 2)

        kv_full[my_pos] = kv_ref[...]

        for h in range(N_DEV - 1):
            origin = lax.rem(my_pos - h + N_DEV, N_DEV)
            rdma = pltpu.make_async_remote_copy(
                src_ref=kv_full.at[origin],
                dst_ref=kv_full.at[origin],
                send_sem=send_sems.at[h],
                recv_sem=recv_sems.at[h],
                device_id=(right,),
                device_id_type=pl.DeviceIdType.MESH,
            )
            rdma.start()
            rdma.wait()

        qi = lax.broadcasted_iota(jnp.int32, (Sq, Skv), 0) // 64
        kj = lax.broadcasted_iota(jnp.int32, (Sq, Skv), 1) // 64
        mask = (qi == kj) | (kj == 0) | (lax.rem(qi + kj, 3) == 0)

        for b in range(B):
            xb = x_ref[b]
            acc = jnp.zeros((Sq, E), jnp.float32)
            for h in range(Hq):
                q = lax.dot(xb, wq_ref[h], preferred_element_type=jnp.float32)
                q = q.astype(jnp.bfloat16)
                s_parts = [
                    lax.dot_general(
                        q, kv_full[d, 0, b, h],
                        (((1,), (0,)), ((), ())),
                        preferred_element_type=jnp.float32,
                    )
                    for d in range(N_DEV)
                ]
                s = jnp.concatenate(s_parts, axis=1) * 0.125
                s = jnp.where(mask, s, -1e9)
                m = jnp.max(s, axis=1, keepdims=True)
                w = jnp.exp(s - m)
                w = (w / jnp.sum(w, axis=1, keepdims=True)).astype(jnp.bfloat16)
                ctx = jnp.zeros((Sq, Dh), jnp.float32)
                for d in range(N_DEV):
                    ctx = ctx + lax.dot_general(
                        w[:, d * Skv_sh:(d + 1) * Skv_sh], kv_full[d, 1, b, h],
                        (((1,), (1,)), ((), ())),
                        preferred_element_type=jnp.float32,
                    )
                acc = acc + lax.dot(
                    ctx.astype(jnp.bfloat16), wo_ref[h * Dh:(h + 1) * Dh, :],
                    preferred_element_type=jnp.float32,
                )
            out_ref[b] = acc

    return pl.pallas_call(
        body,
        out_shape=jax.ShapeDtypeStruct((B, Sq, E), jnp.float32),
        in_specs=[
            pl.BlockSpec(memory_space=pltpu.VMEM),
            pl.BlockSpec(memory_space=pltpu.VMEM),
            pl.BlockSpec(memory_space=pltpu.VMEM),
            pl.BlockSpec(memory_space=pltpu.VMEM),
        ],
        out_specs=pl.BlockSpec(memory_space=pltpu.VMEM),
        scratch_shapes=[
            pltpu.VMEM((N_DEV, 2, B, Hq, Dh, Skv_sh), jnp.bfloat16),
            pltpu.SemaphoreType.DMA((N_DEV - 1,)),
            pltpu.SemaphoreType.DMA((N_DEV - 1,)),
        ],
        compiler_params=pltpu.CompilerParams(collective_id=0),
    )(x_b, wq_r, kv, wo_b)


# device time: 16347 ns/iter; 1.6057x vs baseline; 1.6057x over previous
import jax
import jax.numpy as jnp
from jax import lax
from jax.experimental import pallas as pl
from jax.experimental.pallas import tpu as pltpu

N_DEV = 4


def kernel(x, Wq, K_ext, V_ext, Wo):
    B, Sq, E = x.shape
    _, CK, Hq, Dh = K_ext.shape
    Do = Hq * Dh

    x_b = x.astype(jnp.bfloat16)
    wq_r = Wq.reshape(E, Hq, Dh).transpose(1, 0, 2).astype(jnp.bfloat16)
    wo_b = Wo.astype(jnp.bfloat16)
    kt = K_ext.transpose(0, 2, 3, 1).astype(jnp.bfloat16)
    vt = V_ext.transpose(0, 2, 3, 1).astype(jnp.bfloat16)

    def body(x_ref, wq_ref, kt_ref, vt_ref, wo_ref, out_ref,
             ctx_buf, ml_buf, ctx_ssems, ml_ssems, ctx_rsems, ml_rsems):
        my_pos = lax.axis_index("i")
        left = lax.rem(my_pos + N_DEV - 1, N_DEV)
        right = lax.rem(my_pos + 1, N_DEV)
        opp = lax.rem(my_pos + 2, N_DEV)
        sends = ((left, 1), (right, 0), (opp, 2))

        barrier_sem = pltpu.get_barrier_semaphore()
        for nbr, _ in sends:
            pl.semaphore_signal(
                barrier_sem, inc=1,
                device_id=(nbr,), device_id_type=pl.DeviceIdType.MESH,
            )
        pl.semaphore_wait(barrier_sem, 3)

        qi = lax.broadcasted_iota(jnp.int32, (Sq, CK), 0) // 64
        kj = lax.broadcasted_iota(jnp.int32, (Sq, CK), 1) // 64 + my_pos * (CK // 64)
        mask = (qi == kj) | (kj == 0) | (lax.rem(qi + kj, 3) == 0)

        ctx_parts = []
        ml_parts = []
        for b in range(B):
            ctx_h = []
            ml_h = []
            for h in range(Hq):
                q = lax.dot(
                    x_ref[b], wq_ref[h], preferred_element_type=jnp.float32
                ).astype(jnp.bfloat16)
                s = lax.dot(
                    q, kt_ref[b, h], preferred_element_type=jnp.float32
                ) * 0.125
                s = jnp.where(mask, s, -1e9)
                m_col = jnp.max(s, axis=1, keepdims=True)
                p = jnp.exp(s - m_col)
                p = jnp.where(mask, p, 0.0)
                l_col = jnp.sum(p, axis=1, keepdims=True)
                ctx_t = lax.dot_general(
                    vt_ref[b, h], p.astype(jnp.bfloat16),
                    (((1,), (1,)), ((), ())),
                    preferred_element_type=jnp.float32,
                )
                ctx_h.append(ctx_t.astype(jnp.bfloat16))
                ml_h.append(jnp.concatenate(
                    [jnp.transpose(m_col), jnp.transpose(l_col)], axis=0
                ))
            ctx_parts.append(jnp.stack(ctx_h))
            ml_parts.append(jnp.stack(ml_h))
        ctx_buf[my_pos] = jnp.stack(ctx_parts)
        ml_buf[my_pos] = jnp.stack(ml_parts)

        started = []
        for idx, (nbr, rel) in enumerate(sends):
            for buf, ssems, rsems in (
                (ctx_buf, ctx_ssems, ctx_rsems),
                (ml_buf, ml_ssems, ml_rsems),
            ):
                r = pltpu.make_async_remote_copy(
                    src_ref=buf.at[my_pos],
                    dst_ref=buf.at[my_pos],
                    send_sem=ssems.at[idx],
                    recv_sem=rsems.at[rel],
                    device_id=(nbr,),
                    device_id_type=pl.DeviceIdType.MESH,
                )
                r.start()
                started.append(r)

        for rel in range(3):
            for buf, rsems in ((ctx_buf, ctx_rsems), (ml_buf, ml_rsems)):
                w = pltpu.make_async_remote_copy(
                    src_ref=buf.at[0], dst_ref=buf.at[0],
                    send_sem=ctx_ssems.at[0], recv_sem=rsems.at[rel],
                    device_id=(my_pos,), device_id_type=pl.DeviceIdType.MESH,
                )
                w.wait_recv()

        for b in range(B):
            acc_out = jnp.zeros((Sq, E), jnp.float32)
            for h in range(Hq):
                ms = [ml_buf[c, b, h, 0:1, :] for c in range(N_DEV)]
                ls = [ml_buf[c, b, h, 1:2, :] for c in range(N_DEV)]
                M = ms[0]
                for c in range(1, N_DEV):
                    M = jnp.maximum(M, ms[c])
                num = jnp.zeros((Dh, Sq), jnp.float32)
                den = jnp.zeros((1, Sq), jnp.float32)
                for c in range(N_DEV):
                    scale = jnp.exp(ms[c] - M)
                    num = num + ctx_buf[c, b, h].astype(jnp.float32) * scale
                    den = den + ls[c] * scale
                ctx_t = (num / den).astype(jnp.bfloat16)
                acc_out = acc_out + lax.dot_general(
                    ctx_t, wo_ref[h * Dh:(h + 1) * Dh, :],
                    (((0,), (0,)), ((), ())),
                    preferred_element_type=jnp.float32,
                )
            out_ref[b] = acc_out

        for r in started:
            r.wait_send()

    return pl.pallas_call(
        body,
        out_shape=jax.ShapeDtypeStruct((B, Sq, E), jnp.float32),
        in_specs=[
            pl.BlockSpec(memory_space=pltpu.VMEM),
            pl.BlockSpec(memory_space=pltpu.VMEM),
            pl.BlockSpec(memory_space=pltpu.VMEM),
            pl.BlockSpec(memory_space=pltpu.VMEM),
            pl.BlockSpec(memory_space=pltpu.VMEM),
        ],
        out_specs=pl.BlockSpec(memory_space=pltpu.VMEM),
        scratch_shapes=[
            pltpu.VMEM((N_DEV, B, Hq, Dh, Sq), jnp.bfloat16),
            pltpu.VMEM((N_DEV, B, Hq, 2, Sq), jnp.float32),
            pltpu.SemaphoreType.DMA((3,)),
            pltpu.SemaphoreType.DMA((3,)),
            pltpu.SemaphoreType.DMA((3,)),
            pltpu.SemaphoreType.DMA((3,)),
        ],
        compiler_params=pltpu.CompilerParams(collective_id=0),
    )(x_b, wq_r, kt, vt, wo_b)


# device time: 14190 ns/iter; 1.8498x vs baseline; 1.1520x over previous
import jax
import jax.numpy as jnp
from jax import lax
from jax.experimental import pallas as pl
from jax.experimental.pallas import tpu as pltpu

N_DEV = 4
N_PEER = 3


def kernel(x, Wq, K_ext, V_ext, Wo):
    B, Sq, E = x.shape
    _, CK, Hq, Dh = K_ext.shape
    Do = Hq * Dh

    x_b = x.astype(jnp.bfloat16)
    wq_b = Wq.astype(jnp.bfloat16)
    wo_b = Wo.astype(jnp.bfloat16)
    kt = K_ext.transpose(0, 2, 3, 1).astype(jnp.bfloat16)
    vt = V_ext.transpose(0, 2, 3, 1).astype(jnp.bfloat16)

    def body(x_ref, wq_ref, kt_ref, vt_ref, wo_ref, out_ref,
             ctx_send, ml_send, ctx_buf, ml_buf,
             ctx_ssems, ml_ssems, ctx_rsems, ml_rsems):
        my_pos = lax.axis_index("i")
        left = lax.rem(my_pos + N_DEV - 1, N_DEV)
        right = lax.rem(my_pos + 1, N_DEV)
        opp = lax.rem(my_pos + 2, N_DEV)
        sends = ((left, 1), (right, 0), (opp, 2))

        barrier_sem = pltpu.get_barrier_semaphore()
        for nbr, _ in sends:
            pl.semaphore_signal(
                barrier_sem, inc=1,
                device_id=(nbr,), device_id_type=pl.DeviceIdType.MESH,
            )
        pl.semaphore_wait(barrier_sem, N_PEER)

        qi = lax.broadcasted_iota(jnp.int32, (Sq, CK), 0) // 64
        kj = lax.broadcasted_iota(jnp.int32, (Sq, CK), 1) // 64 + my_pos * (CK // 64)
        mask = (qi == kj) | (kj == 0) | (lax.rem(qi + kj, 3) == 0)

        started = []

        def send_batch(b):
            for idx, (nbr, rel) in enumerate(sends):
                for src, dst, ssems, rsems in (
                    (ctx_send, ctx_buf, ctx_ssems, ctx_rsems),
                    (ml_send, ml_buf, ml_ssems, ml_rsems),
                ):
                    r = pltpu.make_async_remote_copy(
                        src_ref=src.at[b],
                        dst_ref=dst.at[rel, b],
                        send_sem=ssems.at[idx, b],
                        recv_sem=rsems.at[rel, b],
                        device_id=(nbr,),
                        device_id_type=pl.DeviceIdType.MESH,
                    )
                    r.start()
                    started.append(r)

        own = []
        for b in range(B):
            q_b = lax.dot(
                x_ref[b], wq_ref[...], preferred_element_type=jnp.float32
            ).astype(jnp.bfloat16)
            ms, ls, cs = [], [], []
            for h in range(Hq):
                s = lax.dot(
                    q_b[:, h * Dh:(h + 1) * Dh], kt_ref[b, h],
                    preferred_element_type=jnp.float32,
                ) * 0.125
                s = jnp.where(mask, s, -1e9)
                m_col = jnp.max(s, axis=1, keepdims=True)
                p = jnp.exp(s - m_col)
                p = jnp.where(mask, p, 0.0)
                l_col = jnp.sum(p, axis=1, keepdims=True)
                ctx_t = lax.dot_general(
                    vt_ref[b, h], p.astype(jnp.bfloat16),
                    (((1,), (1,)), ((), ())),
                    preferred_element_type=jnp.float32,
                )
                ms.append(jnp.transpose(m_col))
                ls.append(jnp.transpose(l_col))
                cs.append(ctx_t)
            m_b = jnp.stack(ms)
            l_b = jnp.stack(ls)
            c_b = jnp.stack(cs)
            own.append((m_b, l_b, c_b))
            ctx_send[b] = c_b.astype(jnp.bfloat16)
            ml_send[b] = jnp.concatenate([m_b, l_b], axis=1)
            send_batch(b)

        for b in range(B):
            M, l, acc = own[b]
            for rel in range(N_PEER):
                for dst, ssems, rsems in (
                    (ctx_buf, ctx_ssems, ctx_rsems),
                    (ml_buf, ml_ssems, ml_rsems),
                ):
                    w = pltpu.make_async_remote_copy(
                        src_ref=dst.at[rel, b], dst_ref=dst.at[rel, b],
                        send_sem=ssems.at[0, 0], recv_sem=rsems.at[rel, b],
                        device_id=(my_pos,), device_id_type=pl.DeviceIdType.MESH,
                    )
                    w.wait_recv()
                ml_r = ml_buf[rel, b]
                m_r = ml_r[:, 0:1, :]
                l_r = ml_r[:, 1:2, :]
                ctx_r = ctx_buf[rel, b].astype(jnp.float32)
                Mn = jnp.maximum(M, m_r)
                sc_old = jnp.exp(M - Mn)
                sc_new = jnp.exp(m_r - Mn)
                acc = acc * sc_old + ctx_r * sc_new
                l = l * sc_old + l_r * sc_new
                M = Mn
            ct = (acc / l).astype(jnp.bfloat16).reshape(Do, Sq)
            out_ref[b] = lax.dot_general(
                ct, wo_ref[...],
                (((0,), (0,)), ((), ())),
                preferred_element_type=jnp.float32,
            )

        for r in started:
            r.wait_send()

    return pl.pallas_call(
        body,
        out_shape=jax.ShapeDtypeStruct((B, Sq, E), jnp.float32),
        in_specs=[pl.BlockSpec(memory_space=pltpu.VMEM)] * 5,
        out_specs=pl.BlockSpec(memory_space=pltpu.VMEM),
        scratch_shapes=[
            pltpu.VMEM((B, Hq, Dh, Sq), jnp.bfloat16),
            pltpu.VMEM((B, Hq, 2, Sq), jnp.float32),
            pltpu.VMEM((N_PEER, B, Hq, Dh, Sq), jnp.bfloat16),
            pltpu.VMEM((N_PEER, B, Hq, 2, Sq), jnp.float32),
            pltpu.SemaphoreType.DMA((N_PEER, B)),
            pltpu.SemaphoreType.DMA((N_PEER, B)),
            pltpu.SemaphoreType.DMA((N_PEER, B)),
            pltpu.SemaphoreType.DMA((N_PEER, B)),
        ],
        compiler_params=pltpu.CompilerParams(collective_id=0),
    )(x_b, wq_b, kt, vt, wo_b)


# device time: 12611 ns/iter; 2.0814x vs baseline; 1.1252x over previous
import jax
import jax.numpy as jnp
from jax import lax
from jax.experimental import pallas as pl
from jax.experimental.pallas import tpu as pltpu

N_DEV = 4
N_PEER = 3


def kernel(x, Wq, K_ext, V_ext, Wo):
    B, Sq, E = x.shape
    _, CK, Hq, Dh = K_ext.shape
    Do = Hq * Dh

    def body(x_ref, wq_ref, k_ref, v_ref, wo_ref, out_ref,
             ctx_send, ml_send, ctx_buf, ml_buf,
             ctx_ssems, ml_ssems, ctx_rsems, ml_rsems):
        my_pos = lax.axis_index("i")
        left = lax.rem(my_pos + N_DEV - 1, N_DEV)
        right = lax.rem(my_pos + 1, N_DEV)
        opp = lax.rem(my_pos + 2, N_DEV)
        sends = ((left, 1), (right, 0), (opp, 2))

        barrier_sem = pltpu.get_barrier_semaphore()
        for nbr, _ in sends:
            pl.semaphore_signal(
                barrier_sem, inc=1,
                device_id=(nbr,), device_id_type=pl.DeviceIdType.MESH,
            )

        qi = lax.broadcasted_iota(jnp.int32, (Sq, CK), 0) // 64
        kj = lax.broadcasted_iota(jnp.int32, (Sq, CK), 1) // 64 + my_pos * (CK // 64)
        mask = (qi == kj) | (kj == 0) | (lax.rem(qi + kj, 3) == 0)

        started = []

        def send_batch(b):
            for idx, (nbr, rel) in enumerate(sends):
                for src, dst, ssems, rsems in (
                    (ctx_send, ctx_buf, ctx_ssems, ctx_rsems),
                    (ml_send, ml_buf, ml_ssems, ml_rsems),
                ):
                    r = pltpu.make_async_remote_copy(
                        src_ref=src.at[b],
                        dst_ref=dst.at[rel, b],
                        send_sem=ssems.at[idx, b],
                        recv_sem=rsems.at[rel, b],
                        device_id=(nbr,),
                        device_id_type=pl.DeviceIdType.MESH,
                    )
                    r.start()
                    started.append(r)

        wq = wq_ref[...].astype(jnp.bfloat16)

        own = []
        for b in range(B):
            q_b = lax.dot(
                x_ref[b].astype(jnp.bfloat16), wq,
                preferred_element_type=jnp.float32,
            ).astype(jnp.bfloat16)
            ms, ls, cs = [], [], []
            for h in range(Hq):
                k_bh = k_ref[b, :, h, :].astype(jnp.bfloat16)
                s = lax.dot_general(
                    q_b[:, h * Dh:(h + 1) * Dh], k_bh,
                    (((1,), (1,)), ((), ())),
                    preferred_element_type=jnp.float32,
                ) * 0.125
                s = jnp.where(mask, s, -1e9)
                m_col = jnp.max(s, axis=1, keepdims=True)
                p = jnp.exp(s - m_col)
                p = jnp.where(mask, p, 0.0)
                l_col = jnp.sum(p, axis=1, keepdims=True)
                v_bh = v_ref[b, :, h, :].astype(jnp.bfloat16)
                ctx_t = lax.dot_general(
                    v_bh, p.astype(jnp.bfloat16),
                    (((0,), (1,)), ((), ())),
                    preferred_element_type=jnp.float32,
                )
                ms.append(jnp.transpose(m_col))
                ls.append(jnp.transpose(l_col))
                cs.append(ctx_t)
            m_b = jnp.stack(ms)
            l_b = jnp.stack(ls)
            c_b = jnp.stack(cs)
            own.append((m_b, l_b, c_b))
            ctx_send[b] = c_b.astype(jnp.bfloat16)
            ml_send[b] = jnp.concatenate([m_b, l_b], axis=1)
            if b == 0:
                pl.semaphore_wait(barrier_sem, N_PEER)
            send_batch(b)

        wo = wo_ref[...].astype(jnp.bfloat16)

        for b in range(B):
            M, l, acc = own[b]
            for rel in range(N_PEER):
                for dst, ssems, rsems in (
                    (ctx_buf, ctx_ssems, ctx_rsems),
                    (ml_buf, ml_ssems, ml_rsems),
                ):
                    w = pltpu.make_async_remote_copy(
                        src_ref=dst.at[rel, b], dst_ref=dst.at[rel, b],
                        send_sem=ssems.at[0, 0], recv_sem=rsems.at[rel, b],
                        device_id=(my_pos,), device_id_type=pl.DeviceIdType.MESH,
                    )
                    w.wait_recv()
                ml_r = ml_buf[rel, b]
                m_r = ml_r[:, 0:1, :]
                l_r = ml_r[:, 1:2, :]
                ctx_r = ctx_buf[rel, b].astype(jnp.float32)
                Mn = jnp.maximum(M, m_r)
                sc_old = jnp.exp(M - Mn)
                sc_new = jnp.exp(m_r - Mn)
                acc = acc * sc_old + ctx_r * sc_new
                l = l * sc_old + l_r * sc_new
                M = Mn
            ct = (acc / l).astype(jnp.bfloat16).reshape(Do, Sq)
            out_ref[b] = lax.dot_general(
                ct, wo,
                (((0,), (0,)), ((), ())),
                preferred_element_type=jnp.float32,
            )

        for r in started:
            r.wait_send()

    return pl.pallas_call(
        body,
        out_shape=jax.ShapeDtypeStruct((B, Sq, E), jnp.float32),
        in_specs=[pl.BlockSpec(memory_space=pltpu.VMEM)] * 5,
        out_specs=pl.BlockSpec(memory_space=pltpu.VMEM),
        scratch_shapes=[
            pltpu.VMEM((B, Hq, Dh, Sq), jnp.bfloat16),
            pltpu.VMEM((B, Hq, 2, Sq), jnp.float32),
            pltpu.VMEM((N_PEER, B, Hq, Dh, Sq), jnp.bfloat16),
            pltpu.VMEM((N_PEER, B, Hq, 2, Sq), jnp.float32),
            pltpu.SemaphoreType.DMA((N_PEER, B)),
            pltpu.SemaphoreType.DMA((N_PEER, B)),
            pltpu.SemaphoreType.DMA((N_PEER, B)),
            pltpu.SemaphoreType.DMA((N_PEER, B)),
        ],
        compiler_params=pltpu.CompilerParams(collective_id=0),
    )(x, Wq, K_ext, V_ext, Wo)
